# Initial kernel scaffold; baseline (speedup 1.0000x reference)
import jax
import jax.numpy as jnp
from jax.experimental import pallas as pl


def kernel(src_emb, src_emb_in, dst_ids):
    raise NotImplementedError("write your pallas kernel here")


# speedup vs baseline: 6.4169x; 6.4169x over previous
import functools

import jax
import jax.numpy as jnp
from jax import lax
from jax.experimental import pallas as pl
from jax.experimental.pallas import tpu as pltpu
from jax.experimental.pallas import tpu_sc as plsc

_N_DST = 10000
_N_EDGES = 320000
_D = 128

_NC = 2
_NS = 16
_COLS = _D // _NC
_EPT = _N_EDGES // _NS
_CHUNK = 800
_SUB = 80
_NSUB = _CHUNK // _SUB
_NSTEPS = _EPT // _CHUNK
_RPT = 640

_mesh = plsc.VectorSubcoreMesh(
    core_axis_name="c", subcore_axis_name="s",
    num_cores=_NC, num_subcores=_NS)


@functools.partial(
    pl.kernel,
    out_type=jax.ShapeDtypeStruct((_N_DST, _D), jnp.float32),
    mesh=_mesh,
    scratch_types=[
        pltpu.VMEM_SHARED((_NS * _RPT, _COLS), jnp.float32),
        pltpu.VMEM((_CHUNK, _COLS), jnp.float32),
        pltpu.VMEM((_NSUB, _SUB), jnp.int32),
    ],
    compiler_params=pltpu.CompilerParams(use_tc_tiling_on_sc=False),
)
def _seg_sum(src_hbm, dst2d_hbm, out_hbm, acc, buf, idx):
    cid = lax.axis_index("c")
    sid = lax.axis_index("s")
    c0 = cid * _COLS

    def preload(k, _):
        r0 = sid * _RPT + k * _SUB
        @pl.when(r0 < _N_DST)
        def _():
            pltpu.sync_copy(
                src_hbm.at[pl.ds(_N_EDGES + r0, _SUB), pl.ds(c0, _COLS)],
                acc.at[pl.ds(r0, _SUB)])
        return ()
    lax.fori_loop(0, _RPT // _SUB, preload, ())
    plsc.subcore_barrier()

    def step(t, _):
        e0 = sid * _EPT + t * _CHUNK
        pltpu.sync_copy(dst2d_hbm.at[pl.ds(e0 // _SUB, _NSUB)], idx)
        pltpu.sync_copy(src_hbm.at[pl.ds(e0, _CHUNK), pl.ds(c0, _COLS)], buf)
        for j in range(_NSUB):
            pltpu.sync_copy(buf.at[pl.ds(j * _SUB, _SUB)],
                            acc.at[idx.at[j]], add=True)
        return ()
    lax.fori_loop(0, _NSTEPS, step, ())
    plsc.subcore_barrier()

    def writeout(k, _):
        r0 = sid * _RPT + k * _SUB
        @pl.when(r0 < _N_DST)
        def _():
            pltpu.sync_copy(acc.at[pl.ds(r0, _SUB)],
                            out_hbm.at[pl.ds(r0, _SUB), pl.ds(c0, _COLS)])
        return ()
    lax.fori_loop(0, _RPT // _SUB, writeout, ())


def kernel(src_emb, src_emb_in, dst_ids):
    del src_emb_in
    dst2d = dst_ids.astype(jnp.int32).reshape(_N_EDGES // _SUB, _SUB)
    return _seg_sum(src_emb, dst2d)


# speedup vs baseline: 9.7386x; 1.5177x over previous
import functools

import jax
import jax.numpy as jnp
from jax import lax
from jax.experimental import pallas as pl
from jax.experimental.pallas import tpu as pltpu
from jax.experimental.pallas import tpu_sc as plsc

_N_DST = 10000
_N_EDGES = 320000
_D = 128

_NC = 2
_NS = 16
_COLS = _D // _NC
_EPT = _N_EDGES // _NS
_CHUNK = 400
_SUB = 80
_NSUB = _CHUNK // _SUB
_NSTEPS = _EPT // _CHUNK
_RPT = 640

_mesh = plsc.VectorSubcoreMesh(
    core_axis_name="c", subcore_axis_name="s",
    num_cores=_NC, num_subcores=_NS)


@functools.partial(
    pl.kernel,
    out_type=jax.ShapeDtypeStruct((_N_DST, _D), jnp.float32),
    mesh=_mesh,
    scratch_types=[
        pltpu.VMEM_SHARED((_NS * _RPT, _COLS), jnp.float32),
        pltpu.VMEM((_CHUNK, _COLS), jnp.float32),
        pltpu.VMEM((_CHUNK, _COLS), jnp.float32),
        pltpu.VMEM((_NSUB, _SUB), jnp.int32),
        pltpu.VMEM((_NSUB, _SUB), jnp.int32),
        pltpu.SemaphoreType.DMA,
        pltpu.SemaphoreType.DMA,
        pltpu.SemaphoreType.DMA,
        pltpu.SemaphoreType.DMA,
        pltpu.SemaphoreType.DMA,
    ],
    compiler_params=pltpu.CompilerParams(use_tc_tiling_on_sc=False),
)
def _seg_sum(src_hbm, dst2d_hbm, out_hbm, acc,
             buf_a, buf_b, idx_a, idx_b,
             sem_ra, sem_rb, sem_ia, sem_ib, sem_s):
    cid = lax.axis_index("c")
    sid = lax.axis_index("s")
    c0 = cid * _COLS
    bufs = (buf_a, buf_b)
    idxs = (idx_a, idx_b)
    sem_r = (sem_ra, sem_rb)
    sem_i = (sem_ia, sem_ib)

    def fire(chunk, b):
        e0 = sid * _EPT + chunk * _CHUNK
        pltpu.async_copy(
            src_hbm.at[pl.ds(e0, _CHUNK), pl.ds(c0, _COLS)], bufs[b], sem_r[b])
        pltpu.async_copy(
            dst2d_hbm.at[pl.ds(e0 // _SUB, _NSUB)], idxs[b], sem_i[b])

    fire(0, 0)
    fire(1, 1)

    def preload(k, _):
        r0 = sid * _RPT + k * _SUB
        @pl.when(r0 < _N_DST)
        def _():
            pltpu.sync_copy(
                src_hbm.at[pl.ds(_N_EDGES + r0, _SUB), pl.ds(c0, _COLS)],
                acc.at[pl.ds(r0, _SUB)])
        return ()
    lax.fori_loop(0, _RPT // _SUB, preload, ())
    plsc.subcore_barrier()

    def process(t, b):
        e0 = sid * _EPT + t * _CHUNK
        pltpu.make_async_copy(
            src_hbm.at[pl.ds(e0, _CHUNK), pl.ds(c0, _COLS)],
            bufs[b], sem_r[b]).wait()
        pltpu.make_async_copy(
            dst2d_hbm.at[pl.ds(e0 // _SUB, _NSUB)], idxs[b], sem_i[b]).wait()
        descs = [
            pltpu.async_copy(bufs[b].at[pl.ds(j * _SUB, _SUB)],
                             acc.at[idxs[b].at[j]], sem_s, add=True)
            for j in range(_NSUB)
        ]
        for d in descs:
            d.wait()
        @pl.when(t + 2 < _NSTEPS)
        def _():
            fire(t + 2, b)

    def step(t, _):
        @pl.when(t % 2 == 0)
        def _():
            process(t, 0)
        @pl.when(t % 2 == 1)
        def _():
            process(t, 1)
        return ()
    lax.fori_loop(0, _NSTEPS, step, ())
    plsc.subcore_barrier()

    def writeout(k, _):
        r0 = sid * _RPT + k * _SUB
        @pl.when(r0 < _N_DST)
        def _():
            pltpu.sync_copy(acc.at[pl.ds(r0, _SUB)],
                            out_hbm.at[pl.ds(r0, _SUB), pl.ds(c0, _COLS)])
        return ()
    lax.fori_loop(0, _RPT // _SUB, writeout, ())


def kernel(src_emb, src_emb_in, dst_ids):
    del src_emb_in
    dst2d = dst_ids.astype(jnp.int32).reshape(_N_EDGES // _SUB, _SUB)
    return _seg_sum(src_emb, dst2d)
